# baseline (speedup 1.0000x reference)
import functools

import jax
import jax.numpy as jnp
from jax import lax
from jax.experimental import pallas as pl
from jax.experimental.pallas import tpu as pltpu
from jax.experimental.pallas import tpu_sc as plsc

_LANES = 16
_NC = 2
_NS = 16
_NSLOT = 2
_SCHUNK = 16
_UNROLL = 8


@functools.lru_cache(maxsize=None)
def _make_sc_add(B, S, D):
    NW = _NC * _NS
    s_per_w = S // NW
    n_iter = s_per_w // _SCHUNK

    mesh = plsc.VectorSubcoreMesh(core_axis_name="c", subcore_axis_name="s")

    xb_types = [pltpu.VMEM((_SCHUNK, D), jnp.float32)
                for _ in range(_NSLOT * B)]
    pb_types = [pltpu.VMEM((_SCHUNK, D), jnp.float32) for _ in range(_NSLOT)]
    sem_types = [pltpu.SemaphoreType.DMA for _ in range(3 * _NSLOT)]

    @functools.partial(
        pl.kernel,
        out_type=jax.ShapeDtypeStruct((B, S, D), jnp.float32),
        mesh=mesh,
        scratch_types=xb_types + pb_types + sem_types,
    )
    def k(x_hbm, pos_hbm, out_hbm, *refs):
        xbs = tuple(tuple(refs[sl * B + b] for b in range(B))
                    for sl in range(_NSLOT))
        pbs = refs[_NSLOT * B:_NSLOT * (B + 1)]
        sems = refs[_NSLOT * (B + 1):]
        sem_x = sems[0:_NSLOT]
        sem_p = sems[_NSLOT:2 * _NSLOT]
        sem_s = sems[2 * _NSLOT:3 * _NSLOT]

        c = lax.axis_index("c")
        s = lax.axis_index("s")
        wid = s * _NC + c
        s_base = wid * s_per_w

        load_h = {}
        store_h = {}

        def issue_loads(it):
            slot = it % _NSLOT
            s0 = pl.multiple_of(s_base + it * _SCHUNK, _SCHUNK)
            hp = pltpu.async_copy(pos_hbm.at[pl.ds(s0, _SCHUNK)],
                                  pbs[slot], sem_p[slot])
            hx = [pltpu.async_copy(x_hbm.at[b, pl.ds(s0, _SCHUNK)],
                                   xbs[slot][b], sem_x[slot])
                  for b in range(B)]
            load_h[it] = (hp, hx)

        def wait_loads(it):
            hp, hx = load_h.pop(it)
            hp.wait()
            for h in hx:
                h.wait()

        def compute(it):
            slot = it % _NSLOT
            xb = xbs[slot]
            pb = pbs[slot]

            def row_body(r, _):
                def col_body(i, _c2):
                    for u in range(_UNROLL):
                        sl = pl.ds((i * _UNROLL + u) * _LANES, _LANES)
                        v = pb[r, sl]
                        for b in range(B):
                            plsc.addupdate(xb[b].at[r, sl], v)
                    return 0

                lax.fori_loop(0, D // (_LANES * _UNROLL), col_body, 0)
                return 0

            lax.fori_loop(0, _SCHUNK, row_body, 0)

        def issue_store(it):
            slot = it % _NSLOT
            s0 = pl.multiple_of(s_base + it * _SCHUNK, _SCHUNK)
            store_h[it] = [pltpu.async_copy(xbs[slot][b],
                                            out_hbm.at[b, pl.ds(s0, _SCHUNK)],
                                            sem_s[slot])
                           for b in range(B)]

        def wait_store(it):
            for h in store_h.pop(it):
                h.wait()

        for it in range(_NSLOT - 1):
            issue_loads(it)
        for it in range(n_iter):
            if it + _NSLOT - 1 < n_iter:
                if it >= 1:
                    wait_store(it - 1)
                issue_loads(it + _NSLOT - 1)
            wait_loads(it)
            compute(it)
            issue_store(it)
        for it in range(n_iter - _NSLOT, n_iter):
            if it >= 0 and it in store_h:
                wait_store(it)

    return k


def kernel(x, pos_emb):
    B, S, D = x.shape
    k = _make_sc_add(B, S, D)
    return k(x, pos_emb)


# speedup vs baseline: 1.1612x; 1.1612x over previous
import functools

import jax
import jax.numpy as jnp
from jax import lax
from jax.experimental import pallas as pl
from jax.experimental.pallas import tpu as pltpu
from jax.experimental.pallas import tpu_sc as plsc

_LANES = 16
_NC = 2
_NS = 16
_NSLOT = 2
_SCHUNK = 16
_UNROLL = 8


@functools.lru_cache(maxsize=None)
def _make_sc_add(B, S, D):
    NW = _NC * _NS
    s_per_w = S // NW
    n_iter = s_per_w // _SCHUNK

    mesh = plsc.VectorSubcoreMesh(core_axis_name="c", subcore_axis_name="s")

    xb_types = [pltpu.VMEM((_SCHUNK, D), jnp.float32)
                for _ in range(_NSLOT * B)]
    pb_types = [pltpu.VMEM((_SCHUNK, D), jnp.float32) for _ in range(_NSLOT)]
    sem_types = [pltpu.SemaphoreType.DMA for _ in range(3 * _NSLOT)]

    @functools.partial(
        pl.kernel,
        out_type=jax.ShapeDtypeStruct((B, S, D), jnp.float32),
        mesh=mesh,
        scratch_types=xb_types + pb_types + sem_types,
    )
    def k(x_hbm, pos_hbm, out_hbm, *refs):
        xbs = tuple(tuple(refs[sl * B + b] for b in range(B))
                    for sl in range(_NSLOT))
        pbs = refs[_NSLOT * B:_NSLOT * (B + 1)]
        sems = refs[_NSLOT * (B + 1):]
        sem_x = sems[0:_NSLOT]
        sem_p = sems[_NSLOT:2 * _NSLOT]
        sem_s = sems[2 * _NSLOT:3 * _NSLOT]

        c = lax.axis_index("c")
        s = lax.axis_index("s")
        wid = s * _NC + c
        s_base = wid * s_per_w

        load_h = {}
        store_h = {}

        def issue_loads(it):
            slot = it % _NSLOT
            s0 = pl.multiple_of(s_base + it * _SCHUNK, _SCHUNK)
            hp = pltpu.async_copy(pos_hbm.at[pl.ds(s0, _SCHUNK)],
                                  pbs[slot], sem_p[slot])
            hx = [pltpu.async_copy(x_hbm.at[b, pl.ds(s0, _SCHUNK)],
                                   xbs[slot][b], sem_x[slot])
                  for b in range(B)]
            load_h[it] = (hp, hx)

        def wait_loads(it):
            with jax.named_scope("wait_loads"):
                hp, hx = load_h.pop(it)
                hp.wait()
                for h in hx:
                    h.wait()

        def compute(it):
            slot = it % _NSLOT
            xb = xbs[slot]
            pb = pbs[slot]

            def row_body(r, _):
                for j in range(D // _LANES):
                    sl = pl.ds(j * _LANES, _LANES)
                    v = pb[r, sl]
                    for b in range(B):
                        plsc.addupdate(xb[b].at[r, sl], v)
                return 0

            with jax.named_scope("add_loop"):
                lax.fori_loop(0, _SCHUNK, row_body, 0)

        def issue_store(it):
            slot = it % _NSLOT
            s0 = pl.multiple_of(s_base + it * _SCHUNK, _SCHUNK)
            store_h[it] = [pltpu.async_copy(xbs[slot][b],
                                            out_hbm.at[b, pl.ds(s0, _SCHUNK)],
                                            sem_s[slot])
                           for b in range(B)]

        def wait_store(it):
            with jax.named_scope("wait_store"):
                for h in store_h.pop(it):
                    h.wait()

        for it in range(_NSLOT - 1):
            issue_loads(it)
        for it in range(n_iter):
            if it + _NSLOT - 1 < n_iter:
                if it >= 1:
                    wait_store(it - 1)
                issue_loads(it + _NSLOT - 1)
            wait_loads(it)
            compute(it)
            issue_store(it)
        for it in range(n_iter - _NSLOT, n_iter):
            if it >= 0 and it in store_h:
                wait_store(it)

    return k


def kernel(x, pos_emb):
    B, S, D = x.shape
    k = _make_sc_add(B, S, D)
    return k(x, pos_emb)


# speedup vs baseline: 1.1865x; 1.0218x over previous
import functools

import jax
import jax.numpy as jnp
from jax import lax
from jax.experimental import pallas as pl
from jax.experimental.pallas import tpu as pltpu
from jax.experimental.pallas import tpu_sc as plsc

_LANES = 16
_NC = 2
_NS = 16
_NSLOT = 4
_DIST = 2
_SCHUNK = 8


@functools.lru_cache(maxsize=None)
def _make_sc_add(B, S, D):
    NW = _NC * _NS
    s_per_w = S // NW
    n_iter = s_per_w // _SCHUNK
    assert n_iter % _NSLOT == 0 and n_iter >= 2 * _NSLOT

    mesh = plsc.VectorSubcoreMesh(core_axis_name="c", subcore_axis_name="s")

    xb_types = [pltpu.VMEM((_SCHUNK, D), jnp.float32)
                for _ in range(_NSLOT * B)]
    pb_types = [pltpu.VMEM((_SCHUNK, D), jnp.float32) for _ in range(_NSLOT)]
    sem_types = [pltpu.SemaphoreType.DMA for _ in range(3 * _NSLOT)]

    @functools.partial(
        pl.kernel,
        out_type=jax.ShapeDtypeStruct((B, S, D), jnp.float32),
        mesh=mesh,
        scratch_types=xb_types + pb_types + sem_types,
    )
    def k(x_hbm, pos_hbm, out_hbm, *refs):
        xbs = tuple(tuple(refs[sl * B + b] for b in range(B))
                    for sl in range(_NSLOT))
        pbs = refs[_NSLOT * B:_NSLOT * (B + 1)]
        sems = refs[_NSLOT * (B + 1):]
        sem_x = sems[0:_NSLOT]
        sem_p = sems[_NSLOT:2 * _NSLOT]
        sem_s = sems[2 * _NSLOT:3 * _NSLOT]

        c = lax.axis_index("c")
        s = lax.axis_index("s")
        wid = s * _NC + c
        s_base = wid * s_per_w

        def issue_loads(it, slot):
            s0 = pl.multiple_of(s_base + it * _SCHUNK, _SCHUNK)
            pltpu.async_copy(pos_hbm.at[pl.ds(s0, _SCHUNK)],
                             pbs[slot], sem_p[slot])
            for b in range(B):
                pltpu.async_copy(x_hbm.at[b, pl.ds(s0, _SCHUNK)],
                                 xbs[slot][b], sem_x[slot])

        def wait_loads(slot):
            pltpu.make_async_copy(pos_hbm.at[pl.ds(0, _SCHUNK)],
                                  pbs[slot], sem_p[slot]).wait()
            for b in range(B):
                pltpu.make_async_copy(x_hbm.at[0, pl.ds(0, _SCHUNK)],
                                      xbs[slot][b], sem_x[slot]).wait()

        def compute(slot):
            xb = xbs[slot]
            pb = pbs[slot]

            def row_body(r, _):
                for j in range(D // _LANES):
                    sl = pl.ds(j * _LANES, _LANES)
                    v = pb[r, sl]
                    for b in range(B):
                        plsc.addupdate(xb[b].at[r, sl], v)
                return 0

            lax.fori_loop(0, _SCHUNK, row_body, 0)

        def issue_store(it, slot):
            s0 = pl.multiple_of(s_base + it * _SCHUNK, _SCHUNK)
            for b in range(B):
                pltpu.async_copy(xbs[slot][b],
                                 out_hbm.at[b, pl.ds(s0, _SCHUNK)],
                                 sem_s[slot])

        def wait_store(slot):
            for b in range(B):
                pltpu.make_async_copy(xbs[slot][b],
                                      out_hbm.at[0, pl.ds(0, _SCHUNK)],
                                      sem_s[slot]).wait()

        def phase(it, slot, first=False, last=False):
            wait_loads(slot)
            compute(slot)
            issue_store(it, slot)
            nslot = (slot + _DIST) % _NSLOT
            if not first:
                wait_store(nslot)
            if not last:
                issue_loads(it + _DIST, nslot)

        for it in range(_DIST):
            issue_loads(it, it % _NSLOT)

        for it in range(_NSLOT):
            phase(it, it % _NSLOT, first=(it < _DIST))

        @pl.loop(_NSLOT, n_iter - _NSLOT, step=_NSLOT)
        def _steady(t0):
            for ph in range(_NSLOT):
                phase(t0 + ph, ph)

        for it in range(n_iter - _NSLOT, n_iter):
            phase(it, it % _NSLOT, last=(it + _DIST >= n_iter))

        for it in range(n_iter - _DIST, n_iter):
            wait_store(it % _NSLOT)

    return k


def kernel(x, pos_emb):
    B, S, D = x.shape
    k = _make_sc_add(B, S, D)
    return k(x, pos_emb)
